# Initial kernel scaffold; baseline (speedup 1.0000x reference)
import jax
import jax.numpy as jnp
from jax.experimental import pallas as pl


def kernel(features, class_ids, stages, shared_protos):
    raise NotImplementedError("write your pallas kernel here")


# speedup vs baseline: 1.1880x; 1.1880x over previous
import functools

import jax
import jax.numpy as jnp
from jax import lax
from jax.experimental import pallas as pl
from jax.experimental.pallas import tpu as pltpu
from jax.experimental.pallas import tpu_sc as plsc

B = 16384
D = 128
S = 8
P = 16
L = 16
NC = 2
NS = 16
NW = NC * NS
BPW = B // NW


def _sc_body(feat_hbm, stages_hbm, protos_hbm, out_hbm,
             protos_v, means_v, stg_v, feat_v):
    wid = lax.axis_index("s") * NC + lax.axis_index("c")
    base = wid * BPW

    pltpu.sync_copy(protos_hbm, protos_v)
    pltpu.sync_copy(stages_hbm.at[pl.ds(base, BPW)], stg_v)
    pltpu.sync_copy(feat_hbm.at[pl.ds(base, BPW)], feat_v)

    scale = 0.5 / P
    for s in range(S):
        for j in range(D // L):
            acc = protos_v[s, 0, pl.ds(j * L, L)]
            for p in range(1, P):
                acc = acc + protos_v[s, p, pl.ds(j * L, L)]
            means_v[pl.ds(s * D + j * L, L)] = acc * scale

    def body(g, carry):
        stv = stg_v[pl.ds(g * L, L)]
        for k in range(L):
            i = g * L + k
            st_off = stv[k] * D
            for j in range(D // L):
                m = means_v[pl.ds(st_off + j * L, L)]
                f = feat_v[i, pl.ds(j * L, L)]
                feat_v[i, pl.ds(j * L, L)] = f + m
        return carry

    lax.fori_loop(0, BPW // L, body, 0)

    pltpu.sync_copy(feat_v, out_hbm.at[pl.ds(base, BPW)])


def kernel(features, class_ids, stages, shared_protos):
    del class_ids
    stages_i32 = stages.astype(jnp.int32)
    mesh = plsc.VectorSubcoreMesh(core_axis_name="c", subcore_axis_name="s")
    k = functools.partial(
        pl.kernel,
        mesh=mesh,
        out_type=jax.ShapeDtypeStruct((B, D), jnp.float32),
        scratch_types=[
            pltpu.VMEM((S, P, D), jnp.float32),
            pltpu.VMEM((S * D,), jnp.float32),
            pltpu.VMEM((BPW,), jnp.int32),
            pltpu.VMEM((BPW, D), jnp.float32),
        ],
    )(_sc_body)
    return k(features, stages_i32, shared_protos)
